# baseline (speedup 1.0000x reference)
import functools

import jax
import jax.numpy as jnp
from jax.experimental import pallas as pl

S, B, D = 512, 32, 1024
L = 512


def _mea_kernel(e0_ref, e1_ref, m_ref, w_ref, out_ref):
    Mb = m_ref[...]
    W = w_ref[...]
    scale_t = jax.lax.dot_general(
        W, Mb,
        dimension_numbers=(((1,), (1,)), ((), ())),
        preferred_element_type=jnp.float32,
    )
    mx = jnp.max(scale_t, axis=1, keepdims=True)
    ex = jnp.exp(scale_t - mx)
    z = jnp.sum(ex, axis=1, keepdims=True)
    alpha_t = ex / z

    e0 = jnp.minimum(e0_ref[0, 0, :], L - 1)
    e1 = jnp.minimum(e1_ref[0, 0, :], S - 1)
    E = e0.shape[0]
    rows = jax.lax.broadcasted_iota(jnp.int32, (L, E), 0)
    cols = jax.lax.broadcasted_iota(jnp.int32, (E, S), 1)
    P = (rows == e0[None, :]).astype(jnp.bfloat16)
    Q = (cols == e1[:, None]).astype(jnp.bfloat16)
    C = jax.lax.dot_general(
        P, Q,
        dimension_numbers=(((1,), (0,)), ((), ())),
        preferred_element_type=jnp.float32,
    )
    hit = C > 0.0

    masked = jnp.where(hit, alpha_t, 0.0)
    sum_e = jnp.sum(masked, axis=1, keepdims=True)
    row_total = jnp.sum(alpha_t, axis=1, keepdims=True)
    denom = sum_e + 1e-10 * (row_total - sum_e) + 1e-10
    out_ref[0, :, :] = jnp.where(hit, alpha_t / denom, 0.0)


@jax.jit
def kernel(M, lengths, edge_ind, W):
    del lengths
    e0 = edge_ind[:, :, 0].astype(jnp.int32).reshape(B, 1, -1)
    e1 = edge_ind[:, :, 1].astype(jnp.int32).reshape(B, 1, -1)
    E = e0.shape[-1]
    M2 = M.reshape(S, B * D).astype(jnp.bfloat16)
    Wb = W.astype(jnp.bfloat16)
    grid = (B,)
    return pl.pallas_call(
        _mea_kernel,
        grid=grid,
        in_specs=[
            pl.BlockSpec((1, 1, E), lambda b: (b, 0, 0)),
            pl.BlockSpec((1, 1, E), lambda b: (b, 0, 0)),
            pl.BlockSpec((S, D), lambda b: (0, b)),
            pl.BlockSpec((L, D), lambda b: (0, 0)),
        ],
        out_specs=pl.BlockSpec((1, L, S), lambda b: (b, 0, 0)),
        out_shape=jax.ShapeDtypeStruct((B, L, S), jnp.float32),
    )(e0, e1, M2, Wb)


# speedup vs baseline: 1.8464x; 1.8464x over previous
import jax
import jax.numpy as jnp
from jax.experimental import pallas as pl
from jax.experimental.pallas import tpu as pltpu

S, B, D = 512, 32, 1024
L = 512


def _mea_kernel(e_ref, m_hbm, w_ref, out_ref, mbuf, sems):
    b = pl.program_id(0)
    slot = jax.lax.rem(b, 2)

    def start(i, sl):
        pltpu.make_async_copy(
            m_hbm.at[:, pl.ds(i, 1), :], mbuf.at[sl], sems.at[sl]
        ).start()

    @pl.when(b == 0)
    def _():
        start(0, 0)

    @pl.when(b + 1 < B)
    def _():
        start(b + 1, jax.lax.rem(b + 1, 2))

    pltpu.make_async_copy(
        m_hbm.at[:, pl.ds(b, 1), :], mbuf.at[slot], sems.at[slot]
    ).wait()
    Mb = mbuf[slot, :, 0, :]

    W = w_ref[...]
    scale_t = jax.lax.dot_general(
        W, Mb,
        dimension_numbers=(((1,), (1,)), ((), ())),
        preferred_element_type=jnp.float32,
    )
    mx = jnp.max(scale_t, axis=1, keepdims=True)
    ex = jnp.exp(scale_t - mx)
    z = jnp.sum(ex, axis=1, keepdims=True)
    alpha_t = ex / z

    ef = e_ref[0]
    e0 = jnp.minimum(ef[:, 0:1], L - 1)
    e1 = jnp.minimum(ef[:, 1:2], S - 1)
    E = ef.shape[0]
    rows = jax.lax.broadcasted_iota(jnp.int32, (E, L), 1)
    cols = jax.lax.broadcasted_iota(jnp.int32, (E, S), 1)
    Pt = (rows == e0).astype(jnp.bfloat16)
    Q = (cols == e1).astype(jnp.bfloat16)
    C = jax.lax.dot_general(
        Pt, Q,
        dimension_numbers=(((0,), (0,)), ((), ())),
        preferred_element_type=jnp.float32,
    )
    hit = C > 0.0

    masked = jnp.where(hit, alpha_t, 0.0)
    sum_e = jnp.sum(masked, axis=1, keepdims=True)
    row_total = jnp.sum(alpha_t, axis=1, keepdims=True)
    denom = sum_e + 1e-10 * (row_total - sum_e) + 1e-10
    out_ref[0, :, :] = jnp.where(hit, alpha_t / denom, 0.0)


@jax.jit
def kernel(M, lengths, edge_ind, W):
    del lengths
    e = edge_ind.astype(jnp.int32)
    E = e.shape[1]
    grid = (B,)
    return pl.pallas_call(
        _mea_kernel,
        grid=grid,
        in_specs=[
            pl.BlockSpec((1, E, 2), lambda b: (b, 0, 0)),
            pl.BlockSpec(memory_space=pl.ANY),
            pl.BlockSpec((L, D), lambda b: (0, 0)),
        ],
        out_specs=pl.BlockSpec((1, L, S), lambda b: (b, 0, 0)),
        out_shape=jax.ShapeDtypeStruct((B, L, S), jnp.float32),
        scratch_shapes=[
            pltpu.VMEM((2, S, 1, D), jnp.float32),
            pltpu.SemaphoreType.DMA((2,)),
        ],
    )(e, M, W)


# speedup vs baseline: 2.2507x; 1.2190x over previous
import jax
import jax.numpy as jnp
from jax.experimental import pallas as pl
from jax.experimental.pallas import tpu as pltpu

S, B, D = 512, 32, 1024
L = 512


NBUF = 4


def _mea_kernel(e_ref, m_hbm, w_ref, out_ref, mbuf, sems):
    b = pl.program_id(0)
    slot = jax.lax.rem(b, NBUF)

    def start(i, sl):
        pltpu.make_async_copy(
            m_hbm.at[:, pl.ds(i, 1), :], mbuf.at[sl], sems.at[sl]
        ).start()

    @pl.when(b == 0)
    def _():
        for i in range(NBUF - 1):
            start(i, i)

    @pl.when(b + NBUF - 1 < B)
    def _():
        start(b + NBUF - 1, jax.lax.rem(b + NBUF - 1, NBUF))

    pltpu.make_async_copy(
        m_hbm.at[:, pl.ds(b, 1), :], mbuf.at[slot], sems.at[slot]
    ).wait()
    Mb = mbuf[slot, :, 0, :]

    W = w_ref[...]
    scale_t = jax.lax.dot_general(
        W, Mb,
        dimension_numbers=(((1,), (1,)), ((), ())),
        preferred_element_type=jnp.float32,
    )
    mx = jnp.max(scale_t, axis=1, keepdims=True)
    ex = jnp.exp(scale_t - mx)
    z = jnp.sum(ex, axis=1, keepdims=True)

    ef = e_ref[0]
    e0 = jnp.minimum(ef[:, 0:1], L - 1)
    e1 = jnp.minimum(ef[:, 1:2], S - 1)
    E = ef.shape[0]
    rows = jax.lax.broadcasted_iota(jnp.int32, (E, L), 1)
    cols = jax.lax.broadcasted_iota(jnp.int32, (E, S), 1)
    Pt = (rows == e0).astype(jnp.bfloat16)
    Q = (cols == e1).astype(jnp.bfloat16)
    C = jax.lax.dot_general(
        Pt, Q,
        dimension_numbers=(((0,), (0,)), ((), ())),
        preferred_element_type=jnp.float32,
    )
    hit = C > 0.0

    sum_e_ex = jnp.sum(jnp.where(hit, ex, 0.0), axis=1, keepdims=True)
    sum_e = sum_e_ex / z
    denom = sum_e + 1e-10 * (1.0 - sum_e) + 1e-10
    r = 1.0 / (z * denom)
    out_ref[0, :, :] = jnp.where(hit, ex * r, 0.0)


@jax.jit
def kernel(M, lengths, edge_ind, W):
    del lengths
    e = edge_ind.astype(jnp.int32)
    E = e.shape[1]
    grid = (B,)
    return pl.pallas_call(
        _mea_kernel,
        grid=grid,
        in_specs=[
            pl.BlockSpec((1, E, 2), lambda b: (b, 0, 0)),
            pl.BlockSpec(memory_space=pl.ANY),
            pl.BlockSpec((L, D), lambda b: (0, 0)),
        ],
        out_specs=pl.BlockSpec((1, L, S), lambda b: (b, 0, 0)),
        out_shape=jax.ShapeDtypeStruct((B, L, S), jnp.float32),
        scratch_shapes=[
            pltpu.VMEM((NBUF, S, 1, D), jnp.float32),
            pltpu.SemaphoreType.DMA((NBUF,)),
        ],
    )(e, M, W)


# speedup vs baseline: 2.3702x; 1.0531x over previous
import jax
import jax.numpy as jnp
from jax.experimental import pallas as pl
from jax.experimental.pallas import tpu as pltpu

S, B, D = 512, 32, 1024
L = 512


NBUF = 4


def _mea_kernel(e_ref, m_hbm, w_ref, out_ref, mbuf, sems):
    b = pl.program_id(0)
    slot = jax.lax.rem(b, NBUF)

    def start(i, sl):
        pltpu.make_async_copy(
            m_hbm.at[:, i, :], mbuf.at[sl], sems.at[sl]
        ).start()

    @pl.when(b == 0)
    def _():
        for i in range(NBUF - 1):
            start(i, i)

    @pl.when(b + NBUF - 1 < B)
    def _():
        start(b + NBUF - 1, jax.lax.rem(b + NBUF - 1, NBUF))

    pltpu.make_async_copy(
        m_hbm.at[:, b, :], mbuf.at[slot], sems.at[slot]
    ).wait()
    Mb = mbuf[slot]

    W = w_ref[...]
    scale_t = jax.lax.dot_general(
        W, Mb,
        dimension_numbers=(((1,), (1,)), ((), ())),
        preferred_element_type=jnp.float32,
    )
    mx = jnp.max(scale_t, axis=1, keepdims=True)
    ex = jnp.exp(scale_t - mx)
    z = jnp.sum(ex, axis=1, keepdims=True)

    ef = e_ref[0]
    e0 = jnp.minimum(ef[:, 0:1], L - 1)
    e1 = jnp.minimum(ef[:, 1:2], S - 1)
    E = ef.shape[0]
    rows = jax.lax.broadcasted_iota(jnp.int32, (E, L), 1)
    cols = jax.lax.broadcasted_iota(jnp.int32, (E, S), 1)
    Pt = (rows == e0).astype(jnp.bfloat16)
    Q = (cols == e1).astype(jnp.bfloat16)
    C = jax.lax.dot_general(
        Pt, Q,
        dimension_numbers=(((0,), (0,)), ((), ())),
        preferred_element_type=jnp.float32,
    )
    hit = C > 0.0

    sum_e_ex = jnp.sum(jnp.where(hit, ex, 0.0), axis=1, keepdims=True)
    sum_e = sum_e_ex / z
    denom = sum_e + 1e-10 * (1.0 - sum_e) + 1e-10
    r = 1.0 / (z * denom)
    out_ref[0, :, :] = jnp.where(hit, ex * r, 0.0)


@jax.jit
def kernel(M, lengths, edge_ind, W):
    del lengths
    e = edge_ind.astype(jnp.int32)
    E = e.shape[1]
    grid = (B,)
    return pl.pallas_call(
        _mea_kernel,
        grid=grid,
        in_specs=[
            pl.BlockSpec((1, E, 2), lambda b: (b, 0, 0)),
            pl.BlockSpec(memory_space=pl.ANY),
            pl.BlockSpec((L, D), lambda b: (0, 0)),
        ],
        out_specs=pl.BlockSpec((1, L, S), lambda b: (b, 0, 0)),
        out_shape=jax.ShapeDtypeStruct((B, L, S), jnp.float32),
        scratch_shapes=[
            pltpu.VMEM((NBUF, S, D), jnp.float32),
            pltpu.SemaphoreType.DMA((NBUF,)),
        ],
    )(e, M, W)
